# Initial kernel scaffold; baseline (speedup 1.0000x reference)
import jax
import jax.numpy as jnp
from jax.experimental import pallas as pl


def kernel(x, edge_index, W1, b1, W2, b2, W3, b3, lin1_W, lin1_b, lin2_W, lin2_b):
    raise NotImplementedError("write your pallas kernel here")


# speedup vs baseline: 37.3905x; 37.3905x over previous
import functools

import jax
import jax.numpy as jnp
from jax import lax
from jax.experimental import pallas as pl
from jax.experimental.pallas import tpu as pltpu
from jax.experimental.pallas import tpu_sc as plsc

N = 10000
E = 320000
D_IN = 128
H = 30
HP = 32
HL = 10
HLP = 16
C = 4

NC = 2
NS = 16
NW = NC * NS
EPW = E // NW
B = 80
CH = EPW // B
NACC = 10240
ZR = NACC // NS

_MESH = plsc.VectorSubcoreMesh(
    core_axis_name="c", subcore_axis_name="s", num_cores=NC, num_subcores=NS)
_SC_PARAMS = pltpu.CompilerParams(use_tc_tiling_on_sc=False)


@functools.partial(
    pl.kernel,
    out_type=jax.ShapeDtypeStruct((NC, NACC, 16), jnp.float32),
    mesh=_MESH,
    scratch_types=[
        pltpu.VMEM((CH, B), jnp.int32),
        pltpu.VMEM((B, 16), jnp.float32),
        pltpu.VMEM_SHARED((NACC, 16), jnp.float32),
    ],
    compiler_params=_SC_PARAMS,
)
def _deg_call(dst_hbm, ones_hbm, zeros_hbm, out_hbm, didx, ones_v, dacc):
    c = lax.axis_index("c")
    s = lax.axis_index("s")
    w = s * NC + c
    pltpu.sync_copy(zeros_hbm.at[pl.ds(s * ZR, ZR)], dacc.at[pl.ds(s * ZR, ZR)])
    pltpu.sync_copy(ones_hbm, ones_v)
    pltpu.sync_copy(dst_hbm.at[w], didx)
    plsc.subcore_barrier()

    def body(j, carry):
        pltpu.sync_copy(ones_v, dacc.at[didx.at[j]], add=True)
        return carry

    lax.fori_loop(0, CH, body, 0)
    plsc.subcore_barrier()
    pltpu.sync_copy(dacc.at[pl.ds(s * ZR, ZR)], out_hbm.at[c, pl.ds(s * ZR, ZR)])


@functools.partial(
    pl.kernel,
    out_type=jax.ShapeDtypeStruct((NC, NACC, HP), jnp.float32),
    mesh=_MESH,
    scratch_types=[
        pltpu.VMEM((CH, B), jnp.int32),
        pltpu.VMEM((CH, B), jnp.int32),
        pltpu.VMEM((B, HP), jnp.float32),
        pltpu.VMEM_SHARED((N, HP), jnp.float32),
        pltpu.VMEM_SHARED((NACC, HP), jnp.float32),
    ],
    compiler_params=_SC_PARAMS,
)
def _spmm_call(h_hbm, src_hbm, dst_hbm, zeros_hbm, out_hbm, sidx, didx, rows,
               h_s, acc):
    c = lax.axis_index("c")
    s = lax.axis_index("s")
    w = s * NC + c

    @pl.when(s < NS - 1)
    def _():
        pltpu.sync_copy(h_hbm.at[pl.ds(s * ZR, ZR)], h_s.at[pl.ds(s * ZR, ZR)])

    @pl.when(s == NS - 1)
    def _():
        pltpu.sync_copy(h_hbm.at[pl.ds((NS - 1) * ZR, N - (NS - 1) * ZR)],
                        h_s.at[pl.ds((NS - 1) * ZR, N - (NS - 1) * ZR)])

    pltpu.sync_copy(zeros_hbm.at[pl.ds(s * ZR, ZR)], acc.at[pl.ds(s * ZR, ZR)])
    pltpu.sync_copy(src_hbm.at[w], sidx)
    pltpu.sync_copy(dst_hbm.at[w], didx)
    plsc.subcore_barrier()

    def body(j, carry):
        pltpu.sync_copy(h_s.at[sidx.at[j]], rows)
        pltpu.sync_copy(rows, acc.at[didx.at[j]], add=True)
        return carry

    lax.fori_loop(0, CH, body, 0)
    plsc.subcore_barrier()
    pltpu.sync_copy(acc.at[pl.ds(s * ZR, ZR)], out_hbm.at[c, pl.ds(s * ZR, ZR)])


BN = 1000
NB = N // BN


def _dinv_of(degb):
    d = degb[0, :, 0:1] + degb[1, :, 0:1] + 1.0
    return lax.rsqrt(d)


def _tc1_body(deg_ref, x_ref, w_ref, o_ref):
    dinv = _dinv_of(deg_ref[...])
    o_ref[...] = jnp.dot(x_ref[...], w_ref[...],
                         preferred_element_type=jnp.float32) * dinv


_tc1 = pl.pallas_call(
    _tc1_body,
    grid=(NB,),
    in_specs=[
        pl.BlockSpec((NC, BN, 16), lambda i: (0, i, 0)),
        pl.BlockSpec((BN, D_IN), lambda i: (i, 0)),
        pl.BlockSpec((D_IN, HP), lambda i: (0, 0)),
    ],
    out_specs=pl.BlockSpec((BN, HP), lambda i: (i, 0)),
    out_shape=jax.ShapeDtypeStruct((N, HP), jnp.float32),
)


def _tc_mid_body(relu, s_ref, hp_ref, deg_ref, b_ref, w_ref, o_ref):
    dinv = _dinv_of(deg_ref[...])
    sv = s_ref[...]
    agg = (sv[0] + sv[1] + hp_ref[...]) * dinv + b_ref[...]
    if relu:
        agg = jnp.maximum(agg, 0.0)
    o_ref[...] = jnp.dot(agg, w_ref[...],
                         preferred_element_type=jnp.float32) * dinv


def _make_tc_mid(relu):
    return pl.pallas_call(
        functools.partial(_tc_mid_body, relu),
        grid=(NB,),
        in_specs=[
            pl.BlockSpec((NC, BN, HP), lambda i: (0, i, 0)),
            pl.BlockSpec((BN, HP), lambda i: (i, 0)),
            pl.BlockSpec((NC, BN, 16), lambda i: (0, i, 0)),
            pl.BlockSpec((1, HP), lambda i: (0, 0)),
            pl.BlockSpec((HP, HP), lambda i: (0, 0)),
        ],
        out_specs=pl.BlockSpec((BN, HP), lambda i: (i, 0)),
        out_shape=jax.ShapeDtypeStruct((N, HP), jnp.float32),
    )


_tc_mid_norelu = _make_tc_mid(False)
_tc_mid_relu = _make_tc_mid(True)


def _tc_head_body(s_ref, hp_ref, deg_ref, b_ref, w1_ref, c1_ref, w2_ref, c2_ref,
                  o_ref):
    dinv = _dinv_of(deg_ref[...])
    sv = s_ref[...]
    h3 = jnp.maximum((sv[0] + sv[1] + hp_ref[...]) * dinv + b_ref[...], 0.0)
    z = jnp.maximum(jnp.dot(h3, w1_ref[...], preferred_element_type=jnp.float32)
                    + c1_ref[...], 0.0)
    o_ref[...] = jnp.dot(z, w2_ref[...],
                         preferred_element_type=jnp.float32) + c2_ref[...]


_tc_head = pl.pallas_call(
    _tc_head_body,
    grid=(NB,),
    in_specs=[
        pl.BlockSpec((NC, BN, HP), lambda i: (0, i, 0)),
        pl.BlockSpec((BN, HP), lambda i: (i, 0)),
        pl.BlockSpec((NC, BN, 16), lambda i: (0, i, 0)),
        pl.BlockSpec((1, HP), lambda i: (0, 0)),
        pl.BlockSpec((HP, HLP), lambda i: (0, 0)),
        pl.BlockSpec((1, HLP), lambda i: (0, 0)),
        pl.BlockSpec((HLP, C), lambda i: (0, 0)),
        pl.BlockSpec((1, C), lambda i: (0, 0)),
    ],
    out_specs=pl.BlockSpec((BN, C), lambda i: (i, 0)),
    out_shape=jax.ShapeDtypeStruct((N, C), jnp.float32),
)


def kernel(x, edge_index, W1, b1, W2, b2, W3, b3, lin1_W, lin1_b, lin2_W, lin2_b):
    src3 = edge_index[0].reshape(NW, CH, B)
    dst3 = edge_index[1].reshape(NW, CH, B)
    zeros32 = jnp.zeros((NACC, HP), jnp.float32)
    zeros16 = jnp.zeros((NACC, 16), jnp.float32)
    ones16 = jnp.ones((B, 16), jnp.float32)

    W1p = jnp.zeros((D_IN, HP), jnp.float32).at[:, :H].set(W1)
    b1p = jnp.zeros((1, HP), jnp.float32).at[0, :H].set(b1)
    W2p = jnp.zeros((HP, HP), jnp.float32).at[:H, :H].set(W2)
    b2p = jnp.zeros((1, HP), jnp.float32).at[0, :H].set(b2)
    W3p = jnp.zeros((HP, HP), jnp.float32).at[:H, :H].set(W3)
    b3p = jnp.zeros((1, HP), jnp.float32).at[0, :H].set(b3)
    l1W = jnp.zeros((HP, HLP), jnp.float32).at[:H, :HL].set(lin1_W)
    l1b = jnp.zeros((1, HLP), jnp.float32).at[0, :HL].set(lin1_b)
    l2W = jnp.zeros((HLP, C), jnp.float32).at[:HL, :].set(lin2_W)
    l2b = lin2_b.reshape(1, C)

    deg2 = _deg_call(dst3, ones16, zeros16)
    h1p = _tc1(deg2, x, W1p)
    S1 = _spmm_call(h1p, src3, dst3, zeros32)
    h2p = _tc_mid_norelu(S1, h1p, deg2, b1p, W2p)
    S2 = _spmm_call(h2p, src3, dst3, zeros32)
    h3p = _tc_mid_relu(S2, h2p, deg2, b2p, W3p)
    S3 = _spmm_call(h3p, src3, dst3, zeros32)
    return _tc_head(S3, h3p, deg2, b3p, l1W, l1b, l2W, l2b)
